# Initial kernel scaffold; baseline (speedup 1.0000x reference)
import jax
import jax.numpy as jnp
from jax.experimental import pallas as pl


def kernel(logits, k):
    raise NotImplementedError("write your pallas kernel here")


# speedup vs baseline: 2.0163x; 2.0163x over previous
import functools

import jax
import jax.numpy as jnp
from jax.experimental import pallas as pl
from jax.experimental.pallas import tpu as pltpu

_TEMPERATURE = 1.0
_MIN_TEMPERATURE = 0.01
_ROWS_PER_BLOCK = 16


def _gumbel_topk_block(k_ref, logits_ref, u_ref, out_ref, rem_ref):
    u = u_ref[...]
    noise = -jnp.log(-jnp.log(u + 1e-20) + 1e-20)
    rem_ref[...] = (logits_ref[...] + noise) * (
        1.0 / max(_TEMPERATURE, _MIN_TEMPERATURE)
    )
    out_ref[...] = jnp.zeros_like(out_ref)

    def body(_, carry):
        rem = rem_ref[...]
        m = jnp.max(rem, axis=-1, keepdims=True)
        e = jnp.exp(rem - m)
        s = jnp.sum(e, axis=-1, keepdims=True)
        p = e * (1.0 / s)
        out_ref[...] += p
        rem_ref[...] = rem - p * 1000000.0
        return carry

    jax.lax.fori_loop(0, k_ref[0], body, 0)
    out_ref[...] = jnp.clip(out_ref[...], 0.0, 1.0)


@functools.partial(jax.jit, static_argnames=())
def _run(logits, k_arr, u):
    n_rows, n_cols = logits.shape
    grid = (n_rows // _ROWS_PER_BLOCK,)
    row_spec = pl.BlockSpec(
        (_ROWS_PER_BLOCK, n_cols), lambda i: (i, 0)
    )
    return pl.pallas_call(
        _gumbel_topk_block,
        grid=grid,
        in_specs=[
            pl.BlockSpec(memory_space=pltpu.SMEM),
            row_spec,
            row_spec,
        ],
        out_specs=row_spec,
        out_shape=jax.ShapeDtypeStruct(logits.shape, logits.dtype),
        scratch_shapes=[pltpu.VMEM((_ROWS_PER_BLOCK, n_cols), jnp.float32)],
    )(k_arr, logits, u)


def kernel(logits, k):
    u = jax.random.uniform(jax.random.key(42), logits.shape, dtype=logits.dtype)
    k_arr = jnp.asarray(k, dtype=jnp.int32).reshape(1)
    return _run(logits, k_arr, u)
